# Initial kernel scaffold; baseline (speedup 1.0000x reference)
import jax
import jax.numpy as jnp
from jax.experimental import pallas as pl


def kernel(x, enc_w1, enc_b1, enc_w2, enc_b2, codebook, dec_w1, dec_b1, dec_w2, dec_b2):
    raise NotImplementedError("write your pallas kernel here")


# speedup vs baseline: 1.0003x; 1.0003x over previous
import jax
import jax.numpy as jnp
from jax.experimental import pallas as pl

B, CIN, L = 16, 64, 4096
HID, LAT, K = 128, 64, 512
T1, T2 = 2048, 1024
F32 = jnp.float32

VQ_BLK = 1024
VQ_GRID = (B * LAT * T2) // LAT // VQ_BLK


def _mm(a, b):
    return jax.lax.dot_general(a, b, (((1,), (0,)), ((), ())),
                               preferred_element_type=F32)


def _mm_t(a, b):
    return jax.lax.dot_general(a, b, (((1,), (1,)), ((), ())),
                               preferred_element_type=F32)


def _shift_r(a):
    return jnp.concatenate([jnp.zeros_like(a[:, :1]), a[:, :-1]], axis=1)


def _shift_l(a):
    return jnp.concatenate([a[:, 1:], jnp.zeros_like(a[:, :1])], axis=1)


def _enc_body(xs_ref, w1_ref, b1_ref, w2_ref, b2_ref, z_ref):
    a0 = xs_ref[0, 0]
    a1 = xs_ref[1, 0]
    a2 = xs_ref[2, 0]
    a3 = xs_ref[3, 0]
    s0 = xs_ref[4, 0]
    s1 = xs_ref[5, 0]
    w1 = w1_ref[...]
    b1 = b1_ref[...]
    w2 = w2_ref[...]
    b2 = b2_ref[...]
    h_e = jax.nn.relu(_mm(w1[0], a0) + _mm(w1[1], a1)
                      + _mm(w1[2], a2) + _mm(w1[3], a3) + b1)
    h_o = jax.nn.relu(_mm(w1[0], a2) + _mm(w1[1], a3)
                      + _mm(w1[2], s0) + _mm(w1[3], s1) + b1)
    z = jax.nn.relu(_mm(w2[0], _shift_r(h_o)) + _mm(w2[1], h_e)
                    + _mm(w2[2], h_o) + _mm(w2[3], _shift_l(h_e)) + b2)
    z_ref[0] = z


def _vq_body(f_ref, cb_ref, q_ref, sse_ref):
    f = f_ref[...]
    cb = cb_ref[...]
    g = _mm_t(f, cb)
    rn = jnp.sum(f * f, axis=1, keepdims=True)
    ones = jnp.ones((1, LAT), F32)
    cn = _mm_t(ones, cb * cb)
    d2 = rn - 2.0 * g + cn
    dist = jnp.sqrt(jnp.maximum(d2, 0.0))
    m = jnp.min(dist, axis=1, keepdims=True)
    iota = jax.lax.broadcasted_iota(jnp.int32, (VQ_BLK, K), 1)
    idx = jnp.min(jnp.where(dist == m, iota, K), axis=1, keepdims=True)
    oh = (iota == idx).astype(F32)
    qb = _mm(oh, cb)
    q_ref[...] = qb
    diff = qb - f

    @pl.when(pl.program_id(0) == 0)
    def _():
        sse_ref[...] = jnp.zeros((1, 1), F32)

    sse_ref[...] += jnp.sum(diff * diff).reshape(1, 1)


def _dec_body(q_ref, v_ref, db1_ref, u_ref, db2_ref, x4_ref, xr_ref, sse_ref):
    q = q_ref[0]
    v = v_ref[...]
    db1 = db1_ref[...]
    u = u_ref[...]
    db2 = db2_ref[...]
    h2e = jax.nn.relu(_mm(v[1], q) + _mm(v[3], _shift_r(q)) + db1)
    h2o = jax.nn.relu(_mm(v[0], _shift_l(q)) + _mm(v[2], q) + db1)
    xr0 = _mm(u[1], h2e) + _mm(u[3], _shift_r(h2o)) + db2
    xr1 = _mm(u[0], h2o) + _mm(u[2], h2e) + db2
    xr2 = _mm(u[1], h2o) + _mm(u[3], h2e) + db2
    xr3 = _mm(u[0], _shift_l(h2e)) + _mm(u[2], h2o) + db2
    xr_ref[0, 0] = xr0
    xr_ref[1, 0] = xr1
    xr_ref[2, 0] = xr2
    xr_ref[3, 0] = xr3
    d0 = xr0 - x4_ref[0, 0]
    d1 = xr1 - x4_ref[1, 0]
    d2 = xr2 - x4_ref[2, 0]
    d3 = xr3 - x4_ref[3, 0]
    s = (jnp.sum(d0 * d0) + jnp.sum(d1 * d1)
         + jnp.sum(d2 * d2) + jnp.sum(d3 * d3))

    @pl.when(pl.program_id(0) == 0)
    def _():
        sse_ref[...] = jnp.zeros((1, 1), F32)

    sse_ref[...] += s.reshape(1, 1)


def kernel(x, enc_w1, enc_b1, enc_w2, enc_b2, codebook,
           dec_w1, dec_b1, dec_w2, dec_b2):
    xp = jnp.pad(x, ((0, 0), (0, 0), (1, 3)))
    xph = xp.reshape(B, CIN, T2 + 1, 4)
    phases = [xph[:, :, 0:T2, p] for p in range(4)]
    shifted = [xph[:, :, 1:T2 + 1, p] for p in range(2)]
    xs = jnp.stack(phases + shifted, axis=0)

    w1s = jnp.transpose(enc_w1, (2, 0, 1))
    w2s = jnp.transpose(enc_w2, (2, 0, 1))
    vs = jnp.transpose(dec_w1, (2, 1, 0))
    us = jnp.transpose(dec_w2, (2, 1, 0))
    b1c = enc_b1[:, None]
    b2c = enc_b2[:, None]
    db1c = dec_b1[:, None]
    db2c = dec_b2[:, None]

    z = pl.pallas_call(
        _enc_body,
        grid=(B,),
        in_specs=[
            pl.BlockSpec((6, 1, CIN, T2), lambda b: (0, b, 0, 0)),
            pl.BlockSpec((4, HID, CIN), lambda b: (0, 0, 0)),
            pl.BlockSpec((HID, 1), lambda b: (0, 0)),
            pl.BlockSpec((4, LAT, HID), lambda b: (0, 0, 0)),
            pl.BlockSpec((LAT, 1), lambda b: (0, 0)),
        ],
        out_specs=pl.BlockSpec((1, LAT, T2), lambda b: (b, 0, 0)),
        out_shape=jax.ShapeDtypeStruct((B, LAT, T2), F32),
    )(xs, w1s, b1c, w2s, b2c)

    flat = z.reshape(-1, LAT)
    q, vq_sse = pl.pallas_call(
        _vq_body,
        grid=(VQ_GRID,),
        in_specs=[
            pl.BlockSpec((VQ_BLK, LAT), lambda i: (i, 0)),
            pl.BlockSpec((K, LAT), lambda i: (0, 0)),
        ],
        out_specs=[
            pl.BlockSpec((VQ_BLK, LAT), lambda i: (i, 0)),
            pl.BlockSpec((1, 1), lambda i: (0, 0)),
        ],
        out_shape=[
            jax.ShapeDtypeStruct((flat.shape[0], LAT), F32),
            jax.ShapeDtypeStruct((1, 1), F32),
        ],
    )(flat, codebook)
    q3 = q.reshape(B, LAT, T2)

    x4 = jnp.transpose(x.reshape(B, CIN, T2, 4), (3, 0, 1, 2))
    xr4, r_sse = pl.pallas_call(
        _dec_body,
        grid=(B,),
        in_specs=[
            pl.BlockSpec((1, LAT, T2), lambda b: (b, 0, 0)),
            pl.BlockSpec((4, HID, LAT), lambda b: (0, 0, 0)),
            pl.BlockSpec((HID, 1), lambda b: (0, 0)),
            pl.BlockSpec((4, CIN, HID), lambda b: (0, 0, 0)),
            pl.BlockSpec((CIN, 1), lambda b: (0, 0)),
            pl.BlockSpec((4, 1, CIN, T2), lambda b: (0, b, 0, 0)),
        ],
        out_specs=[
            pl.BlockSpec((4, 1, CIN, T2), lambda b: (0, b, 0, 0)),
            pl.BlockSpec((1, 1), lambda b: (0, 0)),
        ],
        out_shape=[
            jax.ShapeDtypeStruct((4, B, CIN, T2), F32),
            jax.ShapeDtypeStruct((1, 1), F32),
        ],
    )(q3, vs, db1c, us, db2c, x4)

    x_recon = jnp.transpose(xr4, (1, 2, 3, 0)).reshape(B, CIN, L)
    loss = (r_sse[0, 0] / (B * CIN * L)
            + 1.25 * vq_sse[0, 0] / (B * LAT * T2))
    return (x_recon, loss)


# speedup vs baseline: 2.1588x; 2.1582x over previous
import jax
import jax.numpy as jnp
from jax.experimental import pallas as pl
from jax.experimental.pallas import tpu as pltpu

B, CIN, L = 16, 64, 4096
HID, LAT, K = 128, 64, 512
T1, T2 = 2048, 1024
F32 = jnp.float32

VQ_BLK = 1024
VQ_GRID = (B * LAT * T2) // LAT // VQ_BLK


def _mm(a, b):
    return jax.lax.dot_general(a, b, (((1,), (0,)), ((), ())),
                               preferred_element_type=F32)


def _mm_t(a, b):
    return jax.lax.dot_general(a, b, (((1,), (1,)), ((), ())),
                               preferred_element_type=F32)


def _enc_body(x_ref, w1_ref, b1_ref, w2_ref, b2_ref, z_ref, x_scr, h_scr):
    xt = x_ref[0].T
    x_scr[0:1, :] = jnp.zeros((1, CIN), F32)
    x_scr[1:L + 1, :] = xt
    x_scr[L + 1:, :] = jnp.zeros((7, CIN), F32)
    w1 = w1_ref[...]
    b1 = b1_ref[...]
    h = jnp.zeros((T1, HID), F32)
    for j in range(4):
        h += _mm(x_scr[j:j + L:2, :], w1[j])
    h = jax.nn.relu(h + b1)
    h_scr[0:1, :] = jnp.zeros((1, HID), F32)
    h_scr[1:T1 + 1, :] = h
    h_scr[T1 + 1:, :] = jnp.zeros((7, HID), F32)
    w2 = w2_ref[...]
    b2 = b2_ref[...]
    zt = jnp.zeros((T2, LAT), F32)
    for j in range(4):
        zt += _mm(h_scr[j:j + T1:2, :], w2[j])
    zt = jax.nn.relu(zt + b2)
    z_ref[0] = zt.T


def _vq_body(f_ref, cb_ref, q_ref, sse_ref):
    f = f_ref[...]
    cb = cb_ref[...]
    g = _mm_t(f, cb)
    rn = jnp.sum(f * f, axis=1, keepdims=True)
    ones = jnp.ones((1, LAT), F32)
    cn = _mm_t(ones, cb * cb)
    d2 = rn - 2.0 * g + cn
    dist = jnp.sqrt(jnp.maximum(d2, 0.0))
    m = jnp.min(dist, axis=1, keepdims=True)
    iota = jax.lax.broadcasted_iota(jnp.int32, (VQ_BLK, K), 1)
    idx = jnp.min(jnp.where(dist == m, iota, K), axis=1, keepdims=True)
    oh = (iota == idx).astype(F32)
    qb = _mm(oh, cb)
    q_ref[...] = qb
    diff = qb - f

    @pl.when(pl.program_id(0) == 0)
    def _():
        sse_ref[...] = jnp.zeros((1, 1), F32)

    sse_ref[...] += jnp.sum(diff * diff).reshape(1, 1)


def _dec_body(q_ref, v_ref, db1_ref, u_ref, db2_ref, x_ref, xr_ref, sse_ref,
              q_scr, h2_scr, xr_scr):
    qt = q_ref[0].T
    q_scr[0:1, :] = jnp.zeros((1, LAT), F32)
    q_scr[1:T2 + 1, :] = qt
    q_scr[T2 + 1:, :] = jnp.zeros((7, LAT), F32)
    v = v_ref[...]
    db1 = db1_ref[...]
    h2e = jax.nn.relu(_mm(q_scr[1:T2 + 1, :], v[1])
                      + _mm(q_scr[0:T2, :], v[3]) + db1)
    h2o = jax.nn.relu(_mm(q_scr[2:T2 + 2, :], v[0])
                      + _mm(q_scr[1:T2 + 1, :], v[2]) + db1)
    h2_scr[0:1, :] = jnp.zeros((1, HID), F32)
    h2_scr[1:T1 + 1:2, :] = h2e
    h2_scr[2:T1 + 2:2, :] = h2o
    h2_scr[T1 + 1:, :] = jnp.zeros((7, HID), F32)
    u = u_ref[...]
    db2 = db2_ref[...]
    xre = _mm(h2_scr[1:T1 + 1, :], u[1]) + _mm(h2_scr[0:T1, :], u[3]) + db2
    xro = _mm(h2_scr[2:T1 + 2, :], u[0]) + _mm(h2_scr[1:T1 + 1, :], u[2]) + db2
    xr_scr[0::2, :] = xre
    xr_scr[1::2, :] = xro
    xr = xr_scr[...].T
    xr_ref[0] = xr
    d = xr - x_ref[0]

    @pl.when(pl.program_id(0) == 0)
    def _():
        sse_ref[...] = jnp.zeros((1, 1), F32)

    sse_ref[...] += jnp.sum(d * d).reshape(1, 1)


def kernel(x, enc_w1, enc_b1, enc_w2, enc_b2, codebook,
           dec_w1, dec_b1, dec_w2, dec_b2):
    w1s = jnp.transpose(enc_w1, (2, 1, 0))
    w2s = jnp.transpose(enc_w2, (2, 1, 0))
    vs = jnp.transpose(dec_w1, (2, 0, 1))
    us = jnp.transpose(dec_w2, (2, 0, 1))
    b1r = enc_b1[None, :]
    b2r = enc_b2[None, :]
    db1r = dec_b1[None, :]
    db2r = dec_b2[None, :]

    z = pl.pallas_call(
        _enc_body,
        grid=(B,),
        in_specs=[
            pl.BlockSpec((1, CIN, L), lambda b: (b, 0, 0)),
            pl.BlockSpec((4, CIN, HID), lambda b: (0, 0, 0)),
            pl.BlockSpec((1, HID), lambda b: (0, 0)),
            pl.BlockSpec((4, HID, LAT), lambda b: (0, 0, 0)),
            pl.BlockSpec((1, LAT), lambda b: (0, 0)),
        ],
        out_specs=pl.BlockSpec((1, LAT, T2), lambda b: (b, 0, 0)),
        out_shape=jax.ShapeDtypeStruct((B, LAT, T2), F32),
        scratch_shapes=[
            pltpu.VMEM((L + 8, CIN), F32),
            pltpu.VMEM((T1 + 8, HID), F32),
        ],
    )(x, w1s, b1r, w2s, b2r)

    flat = z.reshape(-1, LAT)
    q, vq_sse = pl.pallas_call(
        _vq_body,
        grid=(VQ_GRID,),
        in_specs=[
            pl.BlockSpec((VQ_BLK, LAT), lambda i: (i, 0)),
            pl.BlockSpec((K, LAT), lambda i: (0, 0)),
        ],
        out_specs=[
            pl.BlockSpec((VQ_BLK, LAT), lambda i: (i, 0)),
            pl.BlockSpec((1, 1), lambda i: (0, 0)),
        ],
        out_shape=[
            jax.ShapeDtypeStruct((flat.shape[0], LAT), F32),
            jax.ShapeDtypeStruct((1, 1), F32),
        ],
    )(flat, codebook)
    q3 = q.reshape(B, LAT, T2)

    xr, r_sse = pl.pallas_call(
        _dec_body,
        grid=(B,),
        in_specs=[
            pl.BlockSpec((1, LAT, T2), lambda b: (b, 0, 0)),
            pl.BlockSpec((4, LAT, HID), lambda b: (0, 0, 0)),
            pl.BlockSpec((1, HID), lambda b: (0, 0)),
            pl.BlockSpec((4, HID, CIN), lambda b: (0, 0, 0)),
            pl.BlockSpec((1, CIN), lambda b: (0, 0)),
            pl.BlockSpec((1, CIN, L), lambda b: (b, 0, 0)),
        ],
        out_specs=[
            pl.BlockSpec((1, CIN, L), lambda b: (b, 0, 0)),
            pl.BlockSpec((1, 1), lambda b: (0, 0)),
        ],
        out_shape=[
            jax.ShapeDtypeStruct((B, CIN, L), F32),
            jax.ShapeDtypeStruct((1, 1), F32),
        ],
        scratch_shapes=[
            pltpu.VMEM((T2 + 8, LAT), F32),
            pltpu.VMEM((T1 + 8, HID), F32),
            pltpu.VMEM((L, CIN), F32),
        ],
    )(q3, vs, db1r, us, db2r, x)

    loss = (r_sse[0, 0] / (B * CIN * L)
            + 1.25 * vq_sse[0, 0] / (B * LAT * T2))
    return (xr, loss)


# speedup vs baseline: 2.2678x; 1.0505x over previous
import jax
import jax.numpy as jnp
from jax.experimental import pallas as pl
from jax.experimental.pallas import tpu as pltpu

B, CIN, L = 16, 64, 4096
HID, LAT, K = 128, 64, 512
T1, T2 = 2048, 1024
F32 = jnp.float32

VQ_BLK = 1024
VQ_GRID = (B * LAT * T2) // LAT // VQ_BLK


def _mm(a, b):
    return jax.lax.dot_general(a, b, (((1,), (0,)), ((), ())),
                               preferred_element_type=F32)


def _mm_t(a, b):
    return jax.lax.dot_general(a, b, (((1,), (1,)), ((), ())),
                               preferred_element_type=F32)


def _enc_body(x_ref, w1_ref, b1_ref, w2_ref, b2_ref, z_ref, x_scr, h_scr):
    xt = x_ref[0].T
    x_scr[0:1, :] = jnp.zeros((1, CIN), F32)
    x_scr[1:L + 1, :] = xt
    x_scr[L + 1:, :] = jnp.zeros((7, CIN), F32)
    w1 = w1_ref[...]
    b1 = b1_ref[...]
    h = jnp.zeros((T1, HID), F32)
    for j in range(4):
        h += _mm(x_scr[j:j + L:2, :], w1[j])
    h = jax.nn.relu(h + b1)
    h_scr[0:1, :] = jnp.zeros((1, HID), F32)
    h_scr[1:T1 + 1, :] = h
    h_scr[T1 + 1:, :] = jnp.zeros((7, HID), F32)
    w2 = w2_ref[...]
    b2 = b2_ref[...]
    zt = jnp.zeros((T2, LAT), F32)
    for j in range(4):
        zt += _mm(h_scr[j:j + T1:2, :], w2[j])
    zt = jax.nn.relu(zt + b2)
    z_ref[0] = zt.T


def _vq_body(f_ref, cb_ref, q_ref, sse_ref):
    f = f_ref[...]
    cb = cb_ref[...]
    g = _mm_t(f, cb)
    ones = jnp.ones((1, LAT), F32)
    cn = _mm_t(ones, cb * cb)
    score = cn - (g + g)
    m = jnp.min(score, axis=1, keepdims=True)
    iota = jax.lax.broadcasted_iota(jnp.int32, (VQ_BLK, K), 1)
    idx = jnp.min(jnp.where(score == m, iota, K), axis=1, keepdims=True)
    oh = (iota == idx).astype(F32)
    qb = _mm(oh, cb)
    q_ref[...] = qb
    diff = qb - f

    @pl.when(pl.program_id(0) == 0)
    def _():
        sse_ref[...] = jnp.zeros((1, 1), F32)

    sse_ref[...] += jnp.sum(diff * diff).reshape(1, 1)


def _dec_body(q_ref, v_ref, db1_ref, u_ref, db2_ref, x_ref, xr_ref, sse_ref,
              q_scr, h2_scr, xr_scr):
    qt = q_ref[0].T
    q_scr[0:1, :] = jnp.zeros((1, LAT), F32)
    q_scr[1:T2 + 1, :] = qt
    q_scr[T2 + 1:, :] = jnp.zeros((7, LAT), F32)
    v = v_ref[...]
    db1 = db1_ref[...]
    h2e = jax.nn.relu(_mm(q_scr[1:T2 + 1, :], v[1])
                      + _mm(q_scr[0:T2, :], v[3]) + db1)
    h2o = jax.nn.relu(_mm(q_scr[2:T2 + 2, :], v[0])
                      + _mm(q_scr[1:T2 + 1, :], v[2]) + db1)
    h2_scr[0:1, :] = jnp.zeros((1, HID), F32)
    h2_scr[1:T1 + 1:2, :] = h2e
    h2_scr[2:T1 + 2:2, :] = h2o
    h2_scr[T1 + 1:, :] = jnp.zeros((7, HID), F32)
    u = u_ref[...]
    db2 = db2_ref[...]
    xre = _mm(h2_scr[1:T1 + 1, :], u[1]) + _mm(h2_scr[0:T1, :], u[3]) + db2
    xro = _mm(h2_scr[2:T1 + 2, :], u[0]) + _mm(h2_scr[1:T1 + 1, :], u[2]) + db2
    xr_scr[0::2, :] = xre
    xr_scr[1::2, :] = xro
    xr = xr_scr[...].T
    xr_ref[0] = xr
    d = xr - x_ref[0]

    @pl.when(pl.program_id(0) == 0)
    def _():
        sse_ref[...] = jnp.zeros((1, 1), F32)

    sse_ref[...] += jnp.sum(d * d).reshape(1, 1)


def kernel(x, enc_w1, enc_b1, enc_w2, enc_b2, codebook,
           dec_w1, dec_b1, dec_w2, dec_b2):
    w1s = jnp.transpose(enc_w1, (2, 1, 0))
    w2s = jnp.transpose(enc_w2, (2, 1, 0))
    vs = jnp.transpose(dec_w1, (2, 0, 1))
    us = jnp.transpose(dec_w2, (2, 0, 1))
    b1r = enc_b1[None, :]
    b2r = enc_b2[None, :]
    db1r = dec_b1[None, :]
    db2r = dec_b2[None, :]

    z = pl.pallas_call(
        _enc_body,
        grid=(B,),
        in_specs=[
            pl.BlockSpec((1, CIN, L), lambda b: (b, 0, 0)),
            pl.BlockSpec((4, CIN, HID), lambda b: (0, 0, 0)),
            pl.BlockSpec((1, HID), lambda b: (0, 0)),
            pl.BlockSpec((4, HID, LAT), lambda b: (0, 0, 0)),
            pl.BlockSpec((1, LAT), lambda b: (0, 0)),
        ],
        out_specs=pl.BlockSpec((1, LAT, T2), lambda b: (b, 0, 0)),
        out_shape=jax.ShapeDtypeStruct((B, LAT, T2), F32),
        scratch_shapes=[
            pltpu.VMEM((L + 8, CIN), F32),
            pltpu.VMEM((T1 + 8, HID), F32),
        ],
    )(x, w1s, b1r, w2s, b2r)

    flat = z.reshape(-1, LAT)
    q, vq_sse = pl.pallas_call(
        _vq_body,
        grid=(VQ_GRID,),
        in_specs=[
            pl.BlockSpec((VQ_BLK, LAT), lambda i: (i, 0)),
            pl.BlockSpec((K, LAT), lambda i: (0, 0)),
        ],
        out_specs=[
            pl.BlockSpec((VQ_BLK, LAT), lambda i: (i, 0)),
            pl.BlockSpec((1, 1), lambda i: (0, 0)),
        ],
        out_shape=[
            jax.ShapeDtypeStruct((flat.shape[0], LAT), F32),
            jax.ShapeDtypeStruct((1, 1), F32),
        ],
    )(flat, codebook)
    q3 = q.reshape(B, LAT, T2)

    xr, r_sse = pl.pallas_call(
        _dec_body,
        grid=(B,),
        in_specs=[
            pl.BlockSpec((1, LAT, T2), lambda b: (b, 0, 0)),
            pl.BlockSpec((4, LAT, HID), lambda b: (0, 0, 0)),
            pl.BlockSpec((1, HID), lambda b: (0, 0)),
            pl.BlockSpec((4, HID, CIN), lambda b: (0, 0, 0)),
            pl.BlockSpec((1, CIN), lambda b: (0, 0)),
            pl.BlockSpec((1, CIN, L), lambda b: (b, 0, 0)),
        ],
        out_specs=[
            pl.BlockSpec((1, CIN, L), lambda b: (b, 0, 0)),
            pl.BlockSpec((1, 1), lambda b: (0, 0)),
        ],
        out_shape=[
            jax.ShapeDtypeStruct((B, CIN, L), F32),
            jax.ShapeDtypeStruct((1, 1), F32),
        ],
        scratch_shapes=[
            pltpu.VMEM((T2 + 8, LAT), F32),
            pltpu.VMEM((T1 + 8, HID), F32),
            pltpu.VMEM((L, CIN), F32),
        ],
    )(q3, vs, db1r, us, db2r, x)

    loss = (r_sse[0, 0] / (B * CIN * L)
            + 1.25 * vq_sse[0, 0] / (B * LAT * T2))
    return (xr, loss)


# speedup vs baseline: 2.5922x; 1.1430x over previous
import jax
import jax.numpy as jnp
from jax.experimental import pallas as pl
from jax.experimental.pallas import tpu as pltpu

B, CIN, L = 16, 64, 4096
HID, LAT, K = 128, 64, 512
T1, T2 = 2048, 1024
NSEG = T2 // LAT
F32 = jnp.float32


def _mm(a, b):
    return jax.lax.dot_general(a, b, (((1,), (0,)), ((), ())),
                               preferred_element_type=F32)


def _mm_t(a, b):
    return jax.lax.dot_general(a, b, (((1,), (1,)), ((), ())),
                               preferred_element_type=F32)


def _body(x_ref, w1_ref, b1_ref, w2_ref, b2_ref, cb_ref, v_ref, db1_ref,
          u_ref, db2_ref, xr_ref, vq_ref, rec_ref,
          x_scr, h_scr, f_scr, q_scr, qp_scr, h2_scr, xr_scr):
    xt = x_ref[0].T
    x_scr[0:1, :] = jnp.zeros((1, CIN), F32)
    x_scr[1:L + 1, :] = xt
    x_scr[L + 1:, :] = jnp.zeros((7, CIN), F32)
    w1 = w1_ref[...]
    h = _mm(x_scr[0:L:2, :], w1[0])
    for j in range(1, 4):
        h += _mm(x_scr[j:j + L:2, :], w1[j])
    h = jax.nn.relu(h + b1_ref[...])
    h_scr[0:1, :] = jnp.zeros((1, HID), F32)
    h_scr[1:T1 + 1, :] = h
    h_scr[T1 + 1:, :] = jnp.zeros((7, HID), F32)
    w2 = w2_ref[...]
    zt = _mm(h_scr[0:T1:2, :], w2[0])
    for j in range(1, 4):
        zt += _mm(h_scr[j:j + T1:2, :], w2[j])
    zt = jax.nn.relu(zt + b2_ref[...])

    for s in range(NSEG):
        f_scr[s::NSEG, :] = zt[LAT * s:LAT * (s + 1), :].T
    f = f_scr[...]
    cb = cb_ref[...]
    g = _mm_t(f, cb)
    ones = jnp.ones((1, LAT), F32)
    cn = _mm_t(ones, cb * cb)
    score = cn - (g + g)
    m = jnp.min(score, axis=1, keepdims=True)
    iota = jax.lax.broadcasted_iota(jnp.int32, (T2, K), 1)
    idx = jnp.min(jnp.where(score == m, iota, K), axis=1, keepdims=True)
    oh = (iota == idx).astype(F32)
    qb = _mm(oh, cb)
    q_scr[...] = qb
    dq = qb - f
    qp_scr[0:1, :] = jnp.zeros((1, LAT), F32)
    for s in range(NSEG):
        qp_scr[1 + LAT * s:1 + LAT * (s + 1), :] = q_scr[s::NSEG, :].T
    qp_scr[T2 + 1:, :] = jnp.zeros((7, LAT), F32)

    v = v_ref[...]
    db1 = db1_ref[...]
    h2e = jax.nn.relu(_mm(qp_scr[1:T2 + 1, :], v[1])
                      + _mm(qp_scr[0:T2, :], v[3]) + db1)
    h2o = jax.nn.relu(_mm(qp_scr[2:T2 + 2, :], v[0])
                      + _mm(qp_scr[1:T2 + 1, :], v[2]) + db1)
    h2_scr[0:1, :] = jnp.zeros((1, HID), F32)
    h2_scr[1:T1 + 1:2, :] = h2e
    h2_scr[2:T1 + 2:2, :] = h2o
    h2_scr[T1 + 1:, :] = jnp.zeros((7, HID), F32)
    u = u_ref[...]
    db2 = db2_ref[...]
    xre = _mm(h2_scr[1:T1 + 1, :], u[1]) + _mm(h2_scr[0:T1, :], u[3]) + db2
    xro = _mm(h2_scr[2:T1 + 2, :], u[0]) + _mm(h2_scr[1:T1 + 1, :], u[2]) + db2
    xr_scr[0::2, :] = xre
    xr_scr[1::2, :] = xro
    xr = xr_scr[...].T
    xr_ref[0] = xr
    d = xr - x_ref[0]

    @pl.when(pl.program_id(0) == 0)
    def _():
        vq_ref[...] = jnp.zeros((1, 1), F32)
        rec_ref[...] = jnp.zeros((1, 1), F32)

    vq_ref[...] += jnp.sum(dq * dq).reshape(1, 1)
    rec_ref[...] += jnp.sum(d * d).reshape(1, 1)


def kernel(x, enc_w1, enc_b1, enc_w2, enc_b2, codebook,
           dec_w1, dec_b1, dec_w2, dec_b2):
    w1s = jnp.transpose(enc_w1, (2, 1, 0))
    w2s = jnp.transpose(enc_w2, (2, 1, 0))
    vs = jnp.transpose(dec_w1, (2, 0, 1))
    us = jnp.transpose(dec_w2, (2, 0, 1))
    b1r = enc_b1[None, :]
    b2r = enc_b2[None, :]
    db1r = dec_b1[None, :]
    db2r = dec_b2[None, :]

    full = lambda *s: pl.BlockSpec(s, lambda b: tuple(0 for _ in s))
    xr, vq_sse, r_sse = pl.pallas_call(
        _body,
        grid=(B,),
        in_specs=[
            pl.BlockSpec((1, CIN, L), lambda b: (b, 0, 0)),
            full(4, CIN, HID),
            full(1, HID),
            full(4, HID, LAT),
            full(1, LAT),
            full(K, LAT),
            full(4, LAT, HID),
            full(1, HID),
            full(4, HID, CIN),
            full(1, CIN),
        ],
        out_specs=[
            pl.BlockSpec((1, CIN, L), lambda b: (b, 0, 0)),
            pl.BlockSpec((1, 1), lambda b: (0, 0)),
            pl.BlockSpec((1, 1), lambda b: (0, 0)),
        ],
        out_shape=[
            jax.ShapeDtypeStruct((B, CIN, L), F32),
            jax.ShapeDtypeStruct((1, 1), F32),
            jax.ShapeDtypeStruct((1, 1), F32),
        ],
        scratch_shapes=[
            pltpu.VMEM((L + 8, CIN), F32),
            pltpu.VMEM((T1 + 8, HID), F32),
            pltpu.VMEM((T2, LAT), F32),
            pltpu.VMEM((T2, LAT), F32),
            pltpu.VMEM((T2 + 8, LAT), F32),
            pltpu.VMEM((T1 + 8, HID), F32),
            pltpu.VMEM((L, CIN), F32),
        ],
    )(x, w1s, b1r, w2s, b2r, codebook, vs, db1r, us, db2r)

    loss = (r_sse[0, 0] / (B * CIN * L)
            + 1.25 * vq_sse[0, 0] / (B * LAT * T2))
    return (xr, loss)


# speedup vs baseline: 2.9690x; 1.1454x over previous
import jax
import jax.numpy as jnp
from jax.experimental import pallas as pl
from jax.experimental.pallas import tpu as pltpu

B, CIN, L = 16, 64, 4096
HID, LAT, K = 128, 64, 512
T1, T2 = 2048, 1024
NSEG = T2 // LAT
BL = 2
F32 = jnp.float32


def _mm(a, b):
    return jax.lax.dot_general(a, b, (((1,), (0,)), ((), ())),
                               preferred_element_type=F32)


def _mm_t(a, b):
    return jax.lax.dot_general(a, b, (((1,), (1,)), ((), ())),
                               preferred_element_type=F32)


def _body(x_ref, w1_ref, b1_ref, w2_ref, b2_ref, cb_ref, v_ref, db1_ref,
          u_ref, db2_ref, xr_ref, vq_ref, rec_ref,
          x_scr, h_scr, f_scr, q_scr, qp_scr, h2_scr, xr_scr):
    w1 = w1_ref[...]
    b1 = b1_ref[...]
    w2 = w2_ref[...]
    b2 = b2_ref[...]
    cb = cb_ref[...]
    v = v_ref[...]
    db1 = db1_ref[...]
    u = u_ref[...]
    db2 = db2_ref[...]
    ones = jnp.ones((1, LAT), F32)
    cn = _mm_t(ones, cb * cb)
    iota = jax.lax.broadcasted_iota(jnp.int32, (T2, K), 1)
    vq_acc = jnp.zeros((1, 1), F32)
    rec_acc = jnp.zeros((1, 1), F32)

    for i in range(BL):
        xt = x_ref[i].T
        x_scr[i, 0:1, :] = jnp.zeros((1, CIN), F32)
        x_scr[i, 1:L + 1, :] = xt
        x_scr[i, L + 1:, :] = jnp.zeros((7, CIN), F32)
        h = _mm(x_scr[i, 0:L:2, :], w1[0])
        for j in range(1, 4):
            h += _mm(x_scr[i, j:j + L:2, :], w1[j])
        h = jax.nn.relu(h + b1)
        h_scr[i, 0:1, :] = jnp.zeros((1, HID), F32)
        h_scr[i, 1:T1 + 1, :] = h
        h_scr[i, T1 + 1:, :] = jnp.zeros((7, HID), F32)
        zt = _mm(h_scr[i, 0:T1:2, :], w2[0])
        for j in range(1, 4):
            zt += _mm(h_scr[i, j:j + T1:2, :], w2[j])
        zt = jax.nn.relu(zt + b2)

        for s in range(NSEG):
            f_scr[i, s::NSEG, :] = zt[LAT * s:LAT * (s + 1), :].T
        f = f_scr[i]
        g = _mm_t(f, cb)
        score = cn - (g + g)
        m = jnp.min(score, axis=1, keepdims=True)
        idx = jnp.min(jnp.where(score == m, iota, K), axis=1, keepdims=True)
        oh = (iota == idx).astype(F32)
        qb = _mm(oh, cb)
        q_scr[i] = qb
        dq = qb - f
        vq_acc += jnp.sum(dq * dq).reshape(1, 1)
        qp_scr[i, 0:1, :] = jnp.zeros((1, LAT), F32)
        for s in range(NSEG):
            qp_scr[i, 1 + LAT * s:1 + LAT * (s + 1), :] = q_scr[i, s::NSEG, :].T
        qp_scr[i, T2 + 1:, :] = jnp.zeros((7, LAT), F32)

        h2e = jax.nn.relu(_mm(qp_scr[i, 1:T2 + 1, :], v[1])
                          + _mm(qp_scr[i, 0:T2, :], v[3]) + db1)
        h2o = jax.nn.relu(_mm(qp_scr[i, 2:T2 + 2, :], v[0])
                          + _mm(qp_scr[i, 1:T2 + 1, :], v[2]) + db1)
        h2_scr[i, 0:1, :] = jnp.zeros((1, HID), F32)
        h2_scr[i, 1:T1 + 1:2, :] = h2e
        h2_scr[i, 2:T1 + 2:2, :] = h2o
        h2_scr[i, T1 + 1:, :] = jnp.zeros((7, HID), F32)
        xre = (_mm(h2_scr[i, 1:T1 + 1, :], u[1])
               + _mm(h2_scr[i, 0:T1, :], u[3]) + db2)
        xro = (_mm(h2_scr[i, 2:T1 + 2, :], u[0])
               + _mm(h2_scr[i, 1:T1 + 1, :], u[2]) + db2)
        de = xre - x_scr[i, 1:L + 1:2, :]
        do = xro - x_scr[i, 2:L + 2:2, :]
        rec_acc += (jnp.sum(de * de) + jnp.sum(do * do)).reshape(1, 1)
        xr_scr[i, 0::2, :] = xre
        xr_scr[i, 1::2, :] = xro
        xr_ref[i] = xr_scr[i].T

    @pl.when(pl.program_id(0) == 0)
    def _():
        vq_ref[...] = jnp.zeros((1, 1), F32)
        rec_ref[...] = jnp.zeros((1, 1), F32)

    vq_ref[...] += vq_acc
    rec_ref[...] += rec_acc


def kernel(x, enc_w1, enc_b1, enc_w2, enc_b2, codebook,
           dec_w1, dec_b1, dec_w2, dec_b2):
    w1s = jnp.transpose(enc_w1, (2, 1, 0))
    w2s = jnp.transpose(enc_w2, (2, 1, 0))
    vs = jnp.transpose(dec_w1, (2, 0, 1))
    us = jnp.transpose(dec_w2, (2, 0, 1))
    b1r = enc_b1[None, :]
    b2r = enc_b2[None, :]
    db1r = dec_b1[None, :]
    db2r = dec_b2[None, :]

    full = lambda *s: pl.BlockSpec(s, lambda b: tuple(0 for _ in s))
    xr, vq_sse, r_sse = pl.pallas_call(
        _body,
        grid=(B // BL,),
        in_specs=[
            pl.BlockSpec((BL, CIN, L), lambda b: (b, 0, 0)),
            full(4, CIN, HID),
            full(1, HID),
            full(4, HID, LAT),
            full(1, LAT),
            full(K, LAT),
            full(4, LAT, HID),
            full(1, HID),
            full(4, HID, CIN),
            full(1, CIN),
        ],
        out_specs=[
            pl.BlockSpec((BL, CIN, L), lambda b: (b, 0, 0)),
            pl.BlockSpec((1, 1), lambda b: (0, 0)),
            pl.BlockSpec((1, 1), lambda b: (0, 0)),
        ],
        out_shape=[
            jax.ShapeDtypeStruct((B, CIN, L), F32),
            jax.ShapeDtypeStruct((1, 1), F32),
            jax.ShapeDtypeStruct((1, 1), F32),
        ],
        scratch_shapes=[
            pltpu.VMEM((BL, L + 8, CIN), F32),
            pltpu.VMEM((BL, T1 + 8, HID), F32),
            pltpu.VMEM((BL, T2, LAT), F32),
            pltpu.VMEM((BL, T2, LAT), F32),
            pltpu.VMEM((BL, T2 + 8, LAT), F32),
            pltpu.VMEM((BL, T1 + 8, HID), F32),
            pltpu.VMEM((BL, L, CIN), F32),
        ],
    )(x, w1s, b1r, w2s, b2r, codebook, vs, db1r, us, db2r)

    loss = (r_sse[0, 0] / (B * CIN * L)
            + 1.25 * vq_sse[0, 0] / (B * LAT * T2))
    return (xr, loss)


# speedup vs baseline: 3.0820x; 1.0381x over previous
import jax
import jax.numpy as jnp
from jax.experimental import pallas as pl
from jax.experimental.pallas import tpu as pltpu

B, CIN, L = 16, 64, 4096
HID, LAT, K = 128, 64, 512
T1, T2 = 2048, 1024
NSEG = T2 // LAT
BL = 4
F32 = jnp.float32


def _mm(a, b):
    return jax.lax.dot_general(a, b, (((1,), (0,)), ((), ())),
                               preferred_element_type=F32)


def _mm_t(a, b):
    return jax.lax.dot_general(a, b, (((1,), (1,)), ((), ())),
                               preferred_element_type=F32)


def _body(x_ref, w1_ref, b1_ref, w2_ref, b2_ref, cb_ref, v_ref, db1_ref,
          u_ref, db2_ref, xr_ref, vq_ref, rec_ref,
          x_scr, h_scr, f_scr, q_scr, qp_scr, h2_scr, xr_scr):
    w1 = w1_ref[...]
    b1 = b1_ref[...]
    w2 = w2_ref[...]
    b2 = b2_ref[...]
    cb = cb_ref[...]
    v = v_ref[...]
    db1 = db1_ref[...]
    u = u_ref[...]
    db2 = db2_ref[...]
    ones = jnp.ones((1, LAT), F32)
    cn = _mm_t(ones, cb * cb)
    iota = jax.lax.broadcasted_iota(jnp.int32, (T2, K), 1)
    vq_acc = jnp.zeros((1, 1), F32)
    rec_acc = jnp.zeros((1, 1), F32)

    for i in range(BL):
        xt = x_ref[i].T
        x_scr[i, 0:1, :] = jnp.zeros((1, CIN), F32)
        x_scr[i, 1:L + 1, :] = xt
        x_scr[i, L + 1:, :] = jnp.zeros((7, CIN), F32)
        h = _mm(x_scr[i, 0:L:2, :], w1[0])
        for j in range(1, 4):
            h += _mm(x_scr[i, j:j + L:2, :], w1[j])
        h = jax.nn.relu(h + b1)
        h_scr[i, 0:1, :] = jnp.zeros((1, HID), F32)
        h_scr[i, 1:T1 + 1, :] = h
        h_scr[i, T1 + 1:, :] = jnp.zeros((7, HID), F32)
        zt = _mm(h_scr[i, 0:T1:2, :], w2[0])
        for j in range(1, 4):
            zt += _mm(h_scr[i, j:j + T1:2, :], w2[j])
        zt = jax.nn.relu(zt + b2)

        for s in range(NSEG):
            f_scr[i, s::NSEG, :] = zt[LAT * s:LAT * (s + 1), :].T
        f = f_scr[i]
        g = _mm_t(f, cb)
        score = cn - (g + g)
        m = jnp.min(score, axis=1, keepdims=True)
        idx = jnp.min(jnp.where(score == m, iota, K), axis=1, keepdims=True)
        oh = (iota == idx).astype(F32)
        qb = _mm(oh, cb)
        q_scr[i] = qb
        dq = qb - f
        vq_acc += jnp.sum(dq * dq).reshape(1, 1)
        qp_scr[i, 0:1, :] = jnp.zeros((1, LAT), F32)
        for s in range(NSEG):
            qp_scr[i, 1 + LAT * s:1 + LAT * (s + 1), :] = q_scr[i, s::NSEG, :].T
        qp_scr[i, T2 + 1:, :] = jnp.zeros((7, LAT), F32)

        h2e = jax.nn.relu(_mm(qp_scr[i, 1:T2 + 1, :], v[1])
                          + _mm(qp_scr[i, 0:T2, :], v[3]) + db1)
        h2o = jax.nn.relu(_mm(qp_scr[i, 2:T2 + 2, :], v[0])
                          + _mm(qp_scr[i, 1:T2 + 1, :], v[2]) + db1)
        h2_scr[i, 0:1, :] = jnp.zeros((1, HID), F32)
        h2_scr[i, 1:T1 + 1:2, :] = h2e
        h2_scr[i, 2:T1 + 2:2, :] = h2o
        h2_scr[i, T1 + 1:, :] = jnp.zeros((7, HID), F32)
        xre = (_mm(h2_scr[i, 1:T1 + 1, :], u[1])
               + _mm(h2_scr[i, 0:T1, :], u[3]) + db2)
        xro = (_mm(h2_scr[i, 2:T1 + 2, :], u[0])
               + _mm(h2_scr[i, 1:T1 + 1, :], u[2]) + db2)
        de = xre - x_scr[i, 1:L + 1:2, :]
        do = xro - x_scr[i, 2:L + 2:2, :]
        rec_acc += (jnp.sum(de * de) + jnp.sum(do * do)).reshape(1, 1)
        xr_scr[i, 0::2, :] = xre
        xr_scr[i, 1::2, :] = xro
        xr_ref[i] = xr_scr[i].T

    @pl.when(pl.program_id(0) == 0)
    def _():
        vq_ref[...] = jnp.zeros((1, 1), F32)
        rec_ref[...] = jnp.zeros((1, 1), F32)

    vq_ref[...] += vq_acc
    rec_ref[...] += rec_acc


def kernel(x, enc_w1, enc_b1, enc_w2, enc_b2, codebook,
           dec_w1, dec_b1, dec_w2, dec_b2):
    w1s = jnp.transpose(enc_w1, (2, 1, 0))
    w2s = jnp.transpose(enc_w2, (2, 1, 0))
    vs = jnp.transpose(dec_w1, (2, 0, 1))
    us = jnp.transpose(dec_w2, (2, 0, 1))
    b1r = enc_b1[None, :]
    b2r = enc_b2[None, :]
    db1r = dec_b1[None, :]
    db2r = dec_b2[None, :]

    full = lambda *s: pl.BlockSpec(s, lambda b: tuple(0 for _ in s))
    xr, vq_sse, r_sse = pl.pallas_call(
        _body,
        grid=(B // BL,),
        in_specs=[
            pl.BlockSpec((BL, CIN, L), lambda b: (b, 0, 0)),
            full(4, CIN, HID),
            full(1, HID),
            full(4, HID, LAT),
            full(1, LAT),
            full(K, LAT),
            full(4, LAT, HID),
            full(1, HID),
            full(4, HID, CIN),
            full(1, CIN),
        ],
        out_specs=[
            pl.BlockSpec((BL, CIN, L), lambda b: (b, 0, 0)),
            pl.BlockSpec((1, 1), lambda b: (0, 0)),
            pl.BlockSpec((1, 1), lambda b: (0, 0)),
        ],
        out_shape=[
            jax.ShapeDtypeStruct((B, CIN, L), F32),
            jax.ShapeDtypeStruct((1, 1), F32),
            jax.ShapeDtypeStruct((1, 1), F32),
        ],
        scratch_shapes=[
            pltpu.VMEM((BL, L + 8, CIN), F32),
            pltpu.VMEM((BL, T1 + 8, HID), F32),
            pltpu.VMEM((BL, T2, LAT), F32),
            pltpu.VMEM((BL, T2, LAT), F32),
            pltpu.VMEM((BL, T2 + 8, LAT), F32),
            pltpu.VMEM((BL, T1 + 8, HID), F32),
            pltpu.VMEM((BL, L, CIN), F32),
        ],
    )(x, w1s, b1r, w2s, b2r, codebook, vs, db1r, us, db2r)

    loss = (r_sse[0, 0] / (B * CIN * L)
            + 1.25 * vq_sse[0, 0] / (B * LAT * T2))
    return (xr, loss)


# speedup vs baseline: 3.5617x; 1.1557x over previous
import jax
import jax.numpy as jnp
from jax.experimental import pallas as pl
from jax.experimental.pallas import tpu as pltpu

B, CIN, L = 16, 64, 4096
HID, LAT, K = 128, 64, 512
T1, T2 = 2048, 1024
NSEG = T2 // LAT
BL = 4
F32 = jnp.float32


def _mm(a, b):
    return jax.lax.dot_general(a, b, (((1,), (0,)), ((), ())),
                               preferred_element_type=F32)


def _mm_t(a, b):
    return jax.lax.dot_general(a, b, (((1,), (1,)), ((), ())),
                               preferred_element_type=F32)


def _body(x_ref, w1_ref, b1_ref, w2_ref, b2_ref, cb_ref, v_ref, db1_ref,
          u_ref, db2_ref, xr_ref, vq_ref, rec_ref,
          x_scr, h_scr, f_scr, q_scr, qp_scr, h2_scr, xr_scr):
    w1 = w1_ref[...]
    b1 = b1_ref[...]
    w2 = w2_ref[...]
    b2 = b2_ref[...]
    cb = cb_ref[...]
    v = v_ref[...]
    db1 = db1_ref[...]
    u = u_ref[...]
    db2 = db2_ref[...]
    ones = jnp.ones((1, LAT), F32)
    cn = _mm_t(ones, cb * cb)
    iota = jax.lax.broadcasted_iota(jnp.int32, (T2, K), 1)
    vq_acc = jnp.zeros((1, 1), F32)
    rec_acc = jnp.zeros((1, 1), F32)

    for i in range(BL):
        xt = x_ref[i].T
        x_scr[i, 0:1, :] = jnp.zeros((1, CIN), F32)
        x_scr[i, 1:L + 1, :] = xt
        x_scr[i, L + 1:, :] = jnp.zeros((7, CIN), F32)
        p01 = jnp.concatenate([x_scr[i, 0:L:2, :], x_scr[i, 1:L + 1:2, :]],
                              axis=1)
        p23 = jnp.concatenate([x_scr[i, 2:L + 2:2, :], x_scr[i, 3:L + 3:2, :]],
                              axis=1)
        h = jax.nn.relu(_mm(p01, w1[0]) + _mm(p23, w1[1]) + b1)
        h_scr[i, 0:1, :] = jnp.zeros((1, HID), F32)
        h_scr[i, 1:T1 + 1, :] = h
        h_scr[i, T1 + 1:, :] = jnp.zeros((7, HID), F32)
        z01 = jnp.concatenate([h_scr[i, 0:T1:2, :], h_scr[i, 1:T1 + 1:2, :]],
                              axis=1)
        z23 = jnp.concatenate([h_scr[i, 2:T1 + 2:2, :], h_scr[i, 3:T1 + 3:2, :]],
                              axis=1)
        zt = jax.nn.relu(_mm(z01, w2[0]) + _mm(z23, w2[1]) + b2)

        for s in range(NSEG):
            f_scr[i, s::NSEG, :] = zt[LAT * s:LAT * (s + 1), :].T
        f = f_scr[i]
        g = _mm_t(f, cb)
        score = cn - (g + g)
        m = jnp.min(score, axis=1, keepdims=True)
        idx = jnp.min(jnp.where(score == m, iota, K), axis=1, keepdims=True)
        oh = (iota == idx).astype(F32)
        qb = _mm(oh, cb)
        q_scr[i] = qb
        dq = qb - f
        vq_acc += jnp.sum(dq * dq).reshape(1, 1)
        qp_scr[i, 0:1, :] = jnp.zeros((1, LAT), F32)
        for s in range(NSEG):
            qp_scr[i, 1 + LAT * s:1 + LAT * (s + 1), :] = q_scr[i, s::NSEG, :].T
        qp_scr[i, T2 + 1:, :] = jnp.zeros((7, LAT), F32)

        qe = jnp.concatenate([qp_scr[i, 1:T2 + 1, :], qp_scr[i, 0:T2, :]],
                             axis=1)
        qo = jnp.concatenate([qp_scr[i, 2:T2 + 2, :], qp_scr[i, 1:T2 + 1, :]],
                             axis=1)
        h2e = jax.nn.relu(_mm(qe, v[0]) + db1)
        h2o = jax.nn.relu(_mm(qo, v[1]) + db1)
        h2_scr[i, 0:1, :] = jnp.zeros((1, HID), F32)
        h2_scr[i, 1:T1 + 1:2, :] = h2e
        h2_scr[i, 2:T1 + 2:2, :] = h2o
        h2_scr[i, T1 + 1:, :] = jnp.zeros((7, HID), F32)
        he2 = jnp.concatenate([h2_scr[i, 1:T1 + 1, :], h2_scr[i, 0:T1, :]],
                              axis=1)
        ho2 = jnp.concatenate([h2_scr[i, 2:T1 + 2, :], h2_scr[i, 1:T1 + 1, :]],
                              axis=1)
        xre = _mm(he2, u[0]) + db2
        xro = _mm(ho2, u[1]) + db2
        de = xre - x_scr[i, 1:L + 1:2, :]
        do = xro - x_scr[i, 2:L + 2:2, :]
        rec_acc += (jnp.sum(de * de) + jnp.sum(do * do)).reshape(1, 1)
        xr_scr[i, 0::2, :] = xre
        xr_scr[i, 1::2, :] = xro
        xr_ref[i] = xr_scr[i].T

    @pl.when(pl.program_id(0) == 0)
    def _():
        vq_ref[...] = jnp.zeros((1, 1), F32)
        rec_ref[...] = jnp.zeros((1, 1), F32)

    vq_ref[...] += vq_acc
    rec_ref[...] += rec_acc


def kernel(x, enc_w1, enc_b1, enc_w2, enc_b2, codebook,
           dec_w1, dec_b1, dec_w2, dec_b2):
    w1t = jnp.transpose(enc_w1, (2, 1, 0))
    w1s = jnp.stack([jnp.concatenate([w1t[0], w1t[1]], axis=0),
                     jnp.concatenate([w1t[2], w1t[3]], axis=0)])
    w2t = jnp.transpose(enc_w2, (2, 1, 0))
    w2s = jnp.stack([jnp.concatenate([w2t[0], w2t[1]], axis=0),
                     jnp.concatenate([w2t[2], w2t[3]], axis=0)])
    vt = jnp.transpose(dec_w1, (2, 0, 1))
    vs = jnp.stack([jnp.concatenate([vt[1], vt[3]], axis=0),
                    jnp.concatenate([vt[0], vt[2]], axis=0)])
    ut = jnp.transpose(dec_w2, (2, 0, 1))
    us = jnp.stack([jnp.concatenate([ut[1], ut[3]], axis=0),
                    jnp.concatenate([ut[0], ut[2]], axis=0)])
    b1r = enc_b1[None, :]
    b2r = enc_b2[None, :]
    db1r = dec_b1[None, :]
    db2r = dec_b2[None, :]

    full = lambda *s: pl.BlockSpec(s, lambda b: tuple(0 for _ in s))
    xr, vq_sse, r_sse = pl.pallas_call(
        _body,
        grid=(B // BL,),
        in_specs=[
            pl.BlockSpec((BL, CIN, L), lambda b: (b, 0, 0)),
            full(2, 2 * CIN, HID),
            full(1, HID),
            full(2, 2 * HID, LAT),
            full(1, LAT),
            full(K, LAT),
            full(2, 2 * LAT, HID),
            full(1, HID),
            full(2, 2 * HID, CIN),
            full(1, CIN),
        ],
        out_specs=[
            pl.BlockSpec((BL, CIN, L), lambda b: (b, 0, 0)),
            pl.BlockSpec((1, 1), lambda b: (0, 0)),
            pl.BlockSpec((1, 1), lambda b: (0, 0)),
        ],
        out_shape=[
            jax.ShapeDtypeStruct((B, CIN, L), F32),
            jax.ShapeDtypeStruct((1, 1), F32),
            jax.ShapeDtypeStruct((1, 1), F32),
        ],
        scratch_shapes=[
            pltpu.VMEM((BL, L + 8, CIN), F32),
            pltpu.VMEM((BL, T1 + 8, HID), F32),
            pltpu.VMEM((BL, T2, LAT), F32),
            pltpu.VMEM((BL, T2, LAT), F32),
            pltpu.VMEM((BL, T2 + 8, LAT), F32),
            pltpu.VMEM((BL, T1 + 8, HID), F32),
            pltpu.VMEM((BL, L, CIN), F32),
        ],
    )(x, w1s, b1r, w2s, b2r, codebook, vs, db1r, us, db2r)

    loss = (r_sse[0, 0] / (B * CIN * L)
            + 1.25 * vq_sse[0, 0] / (B * LAT * T2))
    return (xr, loss)
